# baseline (speedup 1.0000x reference)
import functools

import jax
import jax.numpy as jnp
from jax import lax
from jax.experimental import pallas as pl
from jax.experimental.pallas import tpu as pltpu
from jax.experimental.pallas import tpu_sc as plsc

N = 10000
E = 160000
D = 256
H = 256
P = 64
HOPS = 2

NPAD = 10240
HALF = 5120
ACC_ROWS = 5120
STRIPE = 320
EPT = 10240
EPAD = 16 * EPT
BK = 64
SEGB = 32
NSEG = EPT // (SEGB * BK)
ROWS_PT = EPT // BK
BM = 512
GRID = NPAD // BM


def _spmm_sc(y_pad, gid, loc):
    mesh = plsc.VectorSubcoreMesh(core_axis_name="c", subcore_axis_name="s")

    @functools.partial(
        pl.kernel,
        mesh=mesh,
        compiler_params=pltpu.CompilerParams(use_tc_tiling_on_sc=False),
        out_type=jax.ShapeDtypeStruct((2, ACC_ROWS, H), jnp.float32),
        scratch_types=[
            pltpu.VMEM((SEGB, BK), jnp.int32),
            pltpu.VMEM((SEGB, BK), jnp.int32),
            pltpu.VMEM((BK, H), jnp.float32),
            pltpu.VMEM((BK, H), jnp.float32),
            pltpu.VMEM_SHARED((ACC_ROWS, H), jnp.float32),
            pltpu.SemaphoreType.DMA,
            pltpu.SemaphoreType.DMA,
        ],
    )
    def k(y_hbm, gid_hbm, loc_hbm, out_hbm,
          gid_v, loc_v, rows0, rows1, acc, sem0, sem1):
        c = lax.axis_index("c")
        t = lax.axis_index("s")
        base = t * STRIPE

        def zbody(i, _):
            for q in range(H // 16):
                rows0[i, pl.ds(q * 16, 16)] = jnp.zeros((16,), jnp.float32)
            return _
        lax.fori_loop(0, BK, zbody, None)
        for q in range(STRIPE // BK):
            pltpu.sync_copy(rows0, acc.at[pl.ds(base + q * BK, BK)])

        plsc.subcore_barrier()

        def g_start(j, rows, sem):
            return pltpu.async_copy(
                y_hbm.at[plsc.Indices(gid_v.at[j], ignored_value=-1)],
                rows, sem)

        def g_wait(j, rows, sem):
            pltpu.make_async_copy(
                y_hbm.at[plsc.Indices(gid_v.at[j], ignored_value=-1)],
                rows, sem).wait()

        def s_add(j, rows):
            pltpu.sync_copy(rows,
                            acc.at[plsc.Indices(loc_v.at[j], ignored_value=-1)],
                            add=True)

        def seg_body(g, _):
            row0 = t * ROWS_PT + g * SEGB
            pltpu.sync_copy(gid_hbm.at[c, pl.ds(row0, SEGB)], gid_v)
            pltpu.sync_copy(loc_hbm.at[c, pl.ds(row0, SEGB)], loc_v)

            g_start(0, rows0, sem0)

            def pair(p, _):
                j0 = 2 * p
                g_start(j0 + 1, rows1, sem1)
                g_wait(j0, rows0, sem0)
                s_add(j0, rows0)

                @pl.when(p < SEGB // 2 - 1)
                def _():
                    g_start(j0 + 2, rows0, sem0)
                g_wait(j0 + 1, rows1, sem1)
                s_add(j0 + 1, rows1)
                return _
            lax.fori_loop(0, SEGB // 2, pair, None)
            return _
        lax.fori_loop(0, NSEG, seg_body, None)

        plsc.subcore_barrier()

        for q in range(STRIPE // BK):
            pltpu.sync_copy(acc.at[pl.ds(base + q * BK, BK)],
                            out_hbm.at[c, pl.ds(base + q * BK, BK)])

    return k(y_pad, gid, loc)


ER = 64


def _edge_idx_kernel(s_ref, d_ref, c_ref, g_ref, l_ref, h_ref,
                     sp_ref, dp_ref):
    c = pl.program_id(0)
    i = pl.program_id(1)
    eid = (i * ER + lax.broadcasted_iota(jnp.int32, (ER, 128), 0)) * 128 \
        + lax.broadcasted_iota(jnp.int32, (ER, 128), 1)
    valid = eid < E
    s = jnp.where(valid, s_ref[...], 0)
    d = jnp.where(valid, d_ref[...], -1)
    locd = d - c * HALF
    inb = (locd >= 0) & (locd < HALF)
    g_ref[...] = jnp.where(inb, s, -1)[None]
    l_ref[...] = jnp.where(inb, locd, -1)[None]
    h_ref[...] = jnp.where(d == c_ref[0, 0], s, -1)[None]
    sp_ref[...] = s[None]
    dp_ref[...] = d[None]


def _edge_idx(src, dst, curr_node_id):
    s2 = src.reshape(E // 128, 128)
    d2 = dst.reshape(E // 128, 128)
    nb = EPAD // 128 // ER
    o = pl.pallas_call(
        _edge_idx_kernel,
        grid=(2, nb),
        in_specs=[
            pl.BlockSpec((ER, 128), lambda c, i: (i, 0)),
            pl.BlockSpec((ER, 128), lambda c, i: (i, 0)),
            pl.BlockSpec((1, 1), lambda c, i: (0, 0)),
        ],
        out_specs=[
            pl.BlockSpec((1, ER, 128), lambda c, i: (c, i, 0)),
            pl.BlockSpec((1, ER, 128), lambda c, i: (c, i, 0)),
            pl.BlockSpec((1, ER, 128), lambda c, i: (0, i, 0)),
            pl.BlockSpec((1, ER, 128), lambda c, i: (0, i, 0)),
            pl.BlockSpec((1, ER, 128), lambda c, i: (0, i, 0)),
        ],
        out_shape=[
            jax.ShapeDtypeStruct((2, EPAD // 128, 128), jnp.int32),
            jax.ShapeDtypeStruct((2, EPAD // 128, 128), jnp.int32),
            jax.ShapeDtypeStruct((1, EPAD // 128, 128), jnp.int32),
            jax.ShapeDtypeStruct((1, EPAD // 128, 128), jnp.int32),
            jax.ShapeDtypeStruct((1, EPAD // 128, 128), jnp.int32),
        ],
    )(s2, d2, curr_node_id.reshape(1, 1))
    gid, loc, idx1, sp, dp = o
    return (gid.reshape(2, EPAD // BK, BK), loc.reshape(2, EPAD // BK, BK),
            idx1.reshape(EPAD // BK, BK), sp.reshape(EPAD // BK, BK),
            dp.reshape(EPAD // BK, BK))


MR = NPAD // 16
TSTR = MR // 32
DSTR = MR // 16


DROWS = ROWS_PT // 2


def _deg_sc(dst3, iota_rows):
    mesh = plsc.VectorSubcoreMesh(core_axis_name="c", subcore_axis_name="s")

    @functools.partial(
        pl.kernel,
        mesh=mesh,
        compiler_params=pltpu.CompilerParams(use_tc_tiling_on_sc=False,
                                             needs_layout_passes=False),
        out_type=jax.ShapeDtypeStruct((2, MR, 16), jnp.float32),
        scratch_types=[
            pltpu.VMEM((DROWS, BK), jnp.int32),
            pltpu.VMEM((MR, 16), jnp.float32),
            pltpu.VMEM((5, 128), jnp.int32),
            pltpu.VMEM((DSTR, 16), jnp.float32),
            pltpu.VMEM_SHARED((MR, 16), jnp.float32),
        ],
    )
    def k(dst_hbm, iota_hbm, deg_hbm, d_seg, lhist, iota_v, zrow, shist):
        c = lax.axis_index("c")
        t = lax.axis_index("s")
        ones16 = jnp.ones((16,), jnp.float32)
        zeros16 = jnp.zeros((16,), jnp.float32)

        pltpu.sync_copy(iota_hbm, iota_v)
        w = c * 16 + t
        pltpu.sync_copy(dst_hbm.at[pl.ds(w * DROWS, DROWS)], d_seg)

        def z0(i, _):
            zrow[i, :] = zeros16
            return _
        lax.fori_loop(0, DSTR, z0, None)

        def z1(i, _):
            lhist[i, :] = zeros16
            return _
        lax.fori_loop(0, MR, z1, None)
        pltpu.sync_copy(zrow, shist.at[pl.ds(t * DSTR, DSTR)])
        plsc.subcore_barrier()

        def bodyD(j, _):
            for q in range(BK // 16):
                d16 = d_seg[j, pl.ds(q * 16, 16)]
                md = d16 >= 0
                d16c = jnp.maximum(d16, 0)
                plsc.addupdate_scatter(
                    lhist, [d16c >> 4, d16c & 15], ones16, mask=md)
            return _
        lax.fori_loop(0, DROWS, bodyD, None)

        for b in range(5):
            pltpu.sync_copy(lhist.at[pl.ds(b * 128, 128)],
                            shist.at[plsc.Indices(iota_v.at[b])], add=True)
        plsc.subcore_barrier()

        pltpu.sync_copy(shist.at[pl.ds(t * DSTR, DSTR)],
                        deg_hbm.at[c, pl.ds(t * DSTR, DSTR)])

    return k(dst3, iota_rows)


def _bfs_sc(idx1, src3, dst3, iota_rows, curr_node_id):
    mesh = plsc.VectorSubcoreMesh(core_axis_name="c", subcore_axis_name="s")

    @functools.partial(
        pl.kernel,
        mesh=mesh,
        compiler_params=pltpu.CompilerParams(use_tc_tiling_on_sc=False,
                                             needs_layout_passes=False),
        out_type=jax.ShapeDtypeStruct((MR, 16), jnp.float32),
        scratch_types=[
            pltpu.VMEM((SEGB, BK), jnp.int32),
            pltpu.VMEM((SEGB, BK), jnp.int32),
            pltpu.VMEM((MR, 16), jnp.float32),
            pltpu.VMEM((MR, 16), jnp.float32),
            pltpu.VMEM((5, 128), jnp.int32),
            pltpu.VMEM((DSTR, 16), jnp.float32),
            pltpu.VMEM((16,), jnp.int32),
            pltpu.VMEM_SHARED((MR, 16), jnp.float32),
            pltpu.VMEM_SHARED((MR, 16), jnp.float32),
        ],
    )
    def k(idx1_hbm, src_hbm, dst_hbm, iota_hbm, curr_hbm, nb_hbm,
          a_seg, d_seg, lmask, lhist, iota_v, zrow, curr_v,
          smask1, smask2):
        c = lax.axis_index("c")
        t = lax.axis_index("s")
        ones16 = jnp.ones((16,), jnp.float32)
        zeros16 = jnp.zeros((16,), jnp.float32)

        pltpu.sync_copy(curr_hbm, curr_v)
        pltpu.sync_copy(iota_hbm, iota_v)

        def z0(i, _):
            zrow[i, :] = zeros16
            return _
        lax.fori_loop(0, DSTR, z0, None)

        def z1(i, _):
            lmask[i, :] = zeros16
            return _
        lax.fori_loop(0, MR, z1, None)
        zb = t * DSTR
        pltpu.sync_copy(zrow, smask1.at[pl.ds(zb, DSTR)])
        pltpu.sync_copy(zrow, smask2.at[pl.ds(zb, DSTR)])
        plsc.subcore_barrier()

        def segA(g, _):
            row0 = t * ROWS_PT + g * SEGB
            pltpu.sync_copy(idx1_hbm.at[pl.ds(row0, SEGB)], a_seg)

            def bodyA(j, _):
                for q in range(BK // 16):
                    i16 = a_seg[j, pl.ds(q * 16, 16)]
                    m = i16 >= 0
                    i16c = jnp.maximum(i16, 0)
                    plsc.store_scatter(
                        lmask, [i16c >> 4, i16c & 15], ones16, mask=m)
                return _
            lax.fori_loop(0, SEGB, bodyA, None)
            return _
        lax.fori_loop(0, NSEG, segA, None)

        for b in range(5):
            pltpu.sync_copy(lmask.at[pl.ds(b * 128, 128)],
                            smask1.at[plsc.Indices(iota_v.at[b])], add=True)
        plsc.subcore_barrier()

        cv = curr_v[...]
        pltpu.sync_copy(smask1, lhist)

        def z2(i, _):
            lmask[i, :] = zeros16
            return _
        lax.fori_loop(0, MR, z2, None)

        def segB(g, _):
            row0 = t * ROWS_PT + g * SEGB
            pltpu.sync_copy(src_hbm.at[pl.ds(row0, SEGB)], a_seg)
            pltpu.sync_copy(dst_hbm.at[pl.ds(row0, SEGB)], d_seg)

            def bodyB(j, _):
                for q in range(BK // 16):
                    s16 = a_seg[j, pl.ds(q * 16, 16)]
                    d16 = d_seg[j, pl.ds(q * 16, 16)]
                    d16c = jnp.maximum(d16, 0)
                    mv = plsc.load_gather(lhist, [d16c >> 4, d16c & 15])
                    hit = ((mv > 0.0) | (d16 == cv)) & (d16 >= 0)
                    plsc.store_scatter(
                        lmask, [s16 >> 4, s16 & 15], ones16, mask=hit)
                return _
            lax.fori_loop(0, SEGB, bodyB, None)
            return _
        lax.fori_loop(0, NSEG, segB, None)

        for b in range(5):
            pltpu.sync_copy(lmask.at[pl.ds(b * 128, 128)],
                            smask2.at[plsc.Indices(iota_v.at[b])], add=True)
        plsc.subcore_barrier()

        nbase = (c * 16 + t) * TSTR
        pltpu.sync_copy(smask1.at[pl.ds(nbase, TSTR)], lmask.at[pl.ds(0, TSTR)])
        pltpu.sync_copy(smask2.at[pl.ds(nbase, TSTR)],
                        lmask.at[pl.ds(TSTR, TSTR)])

        def cbody(r, _):
            m1 = lmask[r, :]
            m2 = lmask[TSTR + r, :]
            ids = (nbase + r) * 16 + lax.iota(jnp.int32, 16)
            nb = jnp.where((m1 + m2) > 0.0, 1.0, 0.0)
            nb = jnp.where(ids == cv, 0.0, nb)
            lmask[2 * TSTR + r, :] = nb
            return _
        lax.fori_loop(0, TSTR, cbody, None)
        pltpu.sync_copy(lmask.at[pl.ds(2 * TSTR, TSTR)],
                        nb_hbm.at[pl.ds(nbase, TSTR)])

    return k(idx1, src3, dst3, iota_rows, curr_node_id)


def _mm_scale_kernel(x_ref, w_ref, s_ref, o_ref):
    i = pl.program_id(0)
    rows = i * BM + lax.broadcasted_iota(jnp.int32, (BM, 1), 0)
    xv = jnp.where(rows < N, x_ref[...], 0.0)
    o_ref[...] = jnp.dot(xv, w_ref[...],
                         preferred_element_type=jnp.float32) \
        * lax.rsqrt(s_ref[...] + 1.0)


def _mm_scale(x, w, s):
    return pl.pallas_call(
        _mm_scale_kernel,
        grid=(GRID,),
        in_specs=[
            pl.BlockSpec((BM, D), lambda i: (i, 0)),
            pl.BlockSpec((D, H), lambda i: (0, 0)),
            pl.BlockSpec((BM, 1), lambda i: (i, 0)),
        ],
        out_specs=pl.BlockSpec((BM, H), lambda i: (i, 0)),
        out_shape=jax.ShapeDtypeStruct((NPAD, H), jnp.float32),
    )(x, w, s)


def _agg_stats_kernel(a_ref, y_ref, s_ref, b_ref, z_ref, ps_ref, pq_ref):
    i = pl.program_id(0)
    z = lax.rsqrt(s_ref[...] + 1.0) * (a_ref[...] + y_ref[...]) + b_ref[...]
    rows = i * BM + lax.broadcasted_iota(jnp.int32, (BM, 1), 0)
    z = jnp.where(rows < N, z, 0.0)
    z_ref[...] = z
    ps_ref[...] = jnp.sum(z, axis=0, keepdims=True)[None]
    pq_ref[...] = jnp.sum(z * z, axis=0, keepdims=True)[None]


def _agg_stats(acc, y, dinv, b):
    return pl.pallas_call(
        _agg_stats_kernel,
        grid=(GRID,),
        in_specs=[
            pl.BlockSpec((BM, H), lambda i: (i, 0)),
            pl.BlockSpec((BM, H), lambda i: (i, 0)),
            pl.BlockSpec((BM, 1), lambda i: (i, 0)),
            pl.BlockSpec((1, H), lambda i: (0, 0)),
        ],
        out_specs=[
            pl.BlockSpec((BM, H), lambda i: (i, 0)),
            pl.BlockSpec((1, 1, H), lambda i: (i, 0, 0)),
            pl.BlockSpec((1, 1, H), lambda i: (i, 0, 0)),
        ],
        out_shape=[
            jax.ShapeDtypeStruct((NPAD, H), jnp.float32),
            jax.ShapeDtypeStruct((GRID, 1, H), jnp.float32),
            jax.ShapeDtypeStruct((GRID, 1, H), jnp.float32),
        ],
    )(acc, y, dinv, b)


def _norm_mm_kernel(z_ref, ps_ref, pq_ref, w_ref, s_ref, h_ref, y_ref):
    m = jnp.sum(ps_ref[...], axis=0) / N
    vv = jnp.sum(pq_ref[...], axis=0) / N - m * m
    r = lax.rsqrt(vv + 1e-5)
    hn = jnp.maximum((z_ref[...] - m) * r, 0.0)
    i = pl.program_id(0)
    rows = i * BM + lax.broadcasted_iota(jnp.int32, (BM, 1), 0)
    hn = jnp.where(rows < N, hn, 0.0)
    h_ref[...] = hn
    y_ref[...] = jnp.dot(hn, w_ref[...],
                         preferred_element_type=jnp.float32) \
        * lax.rsqrt(s_ref[...] + 1.0)


def _norm_mm(z, ps, pq, w, s):
    return pl.pallas_call(
        _norm_mm_kernel,
        grid=(GRID,),
        in_specs=[
            pl.BlockSpec((BM, H), lambda i: (i, 0)),
            pl.BlockSpec((GRID, 1, H), lambda i: (0, 0, 0)),
            pl.BlockSpec((GRID, 1, H), lambda i: (0, 0, 0)),
            pl.BlockSpec((H, H), lambda i: (0, 0)),
            pl.BlockSpec((BM, 1), lambda i: (i, 0)),
        ],
        out_specs=[
            pl.BlockSpec((BM, H), lambda i: (i, 0)),
            pl.BlockSpec((BM, H), lambda i: (i, 0)),
        ],
        out_shape=[
            jax.ShapeDtypeStruct((NPAD, H), jnp.float32),
            jax.ShapeDtypeStruct((NPAD, H), jnp.float32),
        ],
    )(z, ps, pq, w, s)


def _mlp_pool_kernel(ha_ref, nw_ref, xc_ref, bmat_ref, b1_ref, v_ref,
                     w2_ref, b2_ref, msk_ref, part_ref, o_ref):
    i = pl.program_id(0)
    c = jnp.dot(xc_ref[...], bmat_ref[...],
                preferred_element_type=jnp.float32) + b1_ref[...]
    s = jnp.maximum(ha_ref[...] + nw_ref[...] * v_ref[...] + c, 0.0)
    sc = (jnp.dot(s, w2_ref[...], preferred_element_type=jnp.float32)
          + b2_ref[...]) * msk_ref[...]
    rows = i * BM + lax.broadcasted_iota(jnp.int32, (BM, 1), 0)
    part = jnp.where(rows < N, part_ref[...], 0.0)
    contrib = jnp.dot(sc.T, part, preferred_element_type=jnp.float32)

    @pl.when(i == 0)
    def _():
        o_ref[...] = jnp.zeros_like(o_ref)
    o_ref[...] += contrib


def _mlp_pool(ha, nw, xc, bmat, b1, v, w2, b2, msk, part):
    return pl.pallas_call(
        _mlp_pool_kernel,
        grid=(GRID,),
        in_specs=[
            pl.BlockSpec((BM, H), lambda i: (i, 0)),
            pl.BlockSpec((BM, 1), lambda i: (i, 0)),
            pl.BlockSpec((1, H), lambda i: (0, 0)),
            pl.BlockSpec((H, H), lambda i: (0, 0)),
            pl.BlockSpec((1, H), lambda i: (0, 0)),
            pl.BlockSpec((1, H), lambda i: (0, 0)),
            pl.BlockSpec((H, 1), lambda i: (0, 0)),
            pl.BlockSpec((1, 1), lambda i: (0, 0)),
            pl.BlockSpec((BM, 1), lambda i: (i, 0)),
            pl.BlockSpec((BM, P), lambda i: (i, 0)),
        ],
        out_specs=pl.BlockSpec((1, P), lambda i: (0, 0)),
        out_shape=jax.ShapeDtypeStruct((1, P), jnp.float32),
    )(ha, nw, xc, bmat, b1, v, w2, b2, msk, part)


def kernel(x, edge_index, curr_node_id, partitions, node_weights,
           W1, b1, W2, b2, lin1_W, lin1_b, lin2_W, lin2_b):
    src = edge_index[0]
    dst = edge_index[1]
    gid, loc, idx1, sp3, dp3 = _edge_idx(src, dst, curr_node_id)

    iota_rows = jnp.arange(MR, dtype=jnp.int32).reshape(5, 128)
    degh = _deg_sc(dp3, iota_rows)
    deg2 = (degh[0] + degh[1]).reshape(NPAD, 1)
    nb2 = _bfs_sc(idx1, sp3, dp3, iota_rows,
                  jnp.broadcast_to(curr_node_id, (16,)))

    def gcn_bn(y_pad, b):
        o = _spmm_sc(y_pad, gid, loc)
        acc = o.reshape(NPAD, H)
        return _agg_stats(acc, y_pad, deg2, b.reshape(1, H))

    zero_s = jnp.zeros((NPAD, 1), jnp.float32)

    y1 = _mm_scale(x, W1, deg2)
    z1, ps1, pq1 = gcn_bn(y1, b1)
    _, y2 = _norm_mm(z1, ps1, pq1, W2, deg2)
    z2, ps2, pq2 = gcn_bn(y2, b2)
    A = lin1_W[:H]
    h_pad, hA = _norm_mm(z2, ps2, pq2, A, zero_s)
    h = h_pad[:N]

    curr = curr_node_id[0]
    x_curr = h[curr_node_id]

    Bmat = lin1_W[H:2 * H]
    v = lin1_W[2 * H].reshape(1, H)
    nw = jnp.pad(node_weights * node_weights[curr], (0, NPAD - N)
                 ).reshape(NPAD, 1)
    msk = nb2.reshape(NPAD, 1)

    partition_scores = _mlp_pool(hA, nw, x_curr, Bmat, lin1_b.reshape(1, H),
                                 v, lin2_W, lin2_b.reshape(1, 1), msk,
                                 partitions)
    return (partition_scores, h)


# speedup vs baseline: 1.0080x; 1.0080x over previous
import functools

import jax
import jax.numpy as jnp
from jax import lax
from jax.experimental import pallas as pl
from jax.experimental.pallas import tpu as pltpu
from jax.experimental.pallas import tpu_sc as plsc

N = 10000
E = 160000
D = 256
H = 256
P = 64
HOPS = 2

NPAD = 10240
HALF = 5120
ACC_ROWS = 5120
STRIPE = 320
EPT = 10240
EPAD = 16 * EPT
BK = 64
SEGB = 40
NSEG = EPT // (SEGB * BK)
ROWS_PT = EPT // BK
BM = 512
GRID = NPAD // BM


def _spmm_sc(y_pad, gid, loc):
    mesh = plsc.VectorSubcoreMesh(core_axis_name="c", subcore_axis_name="s")

    @functools.partial(
        pl.kernel,
        mesh=mesh,
        compiler_params=pltpu.CompilerParams(use_tc_tiling_on_sc=False),
        out_type=jax.ShapeDtypeStruct((2, ACC_ROWS, H), jnp.float32),
        scratch_types=[
            pltpu.VMEM((SEGB, BK), jnp.int32),
            pltpu.VMEM((SEGB, BK), jnp.int32),
            pltpu.VMEM((BK, H), jnp.float32),
            pltpu.VMEM((BK, H), jnp.float32),
            pltpu.VMEM_SHARED((ACC_ROWS, H), jnp.float32),
            pltpu.SemaphoreType.DMA,
            pltpu.SemaphoreType.DMA,
        ],
    )
    def k(y_hbm, gid_hbm, loc_hbm, out_hbm,
          gid_v, loc_v, rows0, rows1, acc, sem0, sem1):
        c = lax.axis_index("c")
        t = lax.axis_index("s")
        base = t * STRIPE

        def zbody(i, _):
            for q in range(H // 16):
                rows0[i, pl.ds(q * 16, 16)] = jnp.zeros((16,), jnp.float32)
            return _
        lax.fori_loop(0, BK, zbody, None)
        for q in range(STRIPE // BK):
            pltpu.sync_copy(rows0, acc.at[pl.ds(base + q * BK, BK)])

        plsc.subcore_barrier()

        def g_start(j, rows, sem):
            return pltpu.async_copy(
                y_hbm.at[plsc.Indices(gid_v.at[j], ignored_value=-1)],
                rows, sem)

        def g_wait(j, rows, sem):
            pltpu.make_async_copy(
                y_hbm.at[plsc.Indices(gid_v.at[j], ignored_value=-1)],
                rows, sem).wait()

        def s_add(j, rows):
            pltpu.sync_copy(rows,
                            acc.at[plsc.Indices(loc_v.at[j], ignored_value=-1)],
                            add=True)

        def seg_body(g, _):
            row0 = t * ROWS_PT + g * SEGB
            pltpu.sync_copy(gid_hbm.at[c, pl.ds(row0, SEGB)], gid_v)
            pltpu.sync_copy(loc_hbm.at[c, pl.ds(row0, SEGB)], loc_v)

            g_start(0, rows0, sem0)

            def pair(p, _):
                j0 = 2 * p
                g_start(j0 + 1, rows1, sem1)
                g_wait(j0, rows0, sem0)
                s_add(j0, rows0)

                @pl.when(p < SEGB // 2 - 1)
                def _():
                    g_start(j0 + 2, rows0, sem0)
                g_wait(j0 + 1, rows1, sem1)
                s_add(j0 + 1, rows1)
                return _
            lax.fori_loop(0, SEGB // 2, pair, None)
            return _
        lax.fori_loop(0, NSEG, seg_body, None)

        plsc.subcore_barrier()

        for q in range(STRIPE // BK):
            pltpu.sync_copy(acc.at[pl.ds(base + q * BK, BK)],
                            out_hbm.at[c, pl.ds(base + q * BK, BK)])

    return k(y_pad, gid, loc)


ER = 64


def _edge_idx_kernel(s_ref, d_ref, c_ref, g_ref, l_ref, h_ref,
                     sp_ref, dp_ref):
    c = pl.program_id(0)
    i = pl.program_id(1)
    eid = (i * ER + lax.broadcasted_iota(jnp.int32, (ER, 128), 0)) * 128 \
        + lax.broadcasted_iota(jnp.int32, (ER, 128), 1)
    valid = eid < E
    s = jnp.where(valid, s_ref[...], 0)
    d = jnp.where(valid, d_ref[...], -1)
    locd = d - c * HALF
    inb = (locd >= 0) & (locd < HALF)
    g_ref[...] = jnp.where(inb, s, -1)[None]
    l_ref[...] = jnp.where(inb, locd, -1)[None]
    h_ref[...] = jnp.where(d == c_ref[0, 0], s, -1)[None]
    sp_ref[...] = s[None]
    dp_ref[...] = d[None]


def _edge_idx(src, dst, curr_node_id):
    s2 = src.reshape(E // 128, 128)
    d2 = dst.reshape(E // 128, 128)
    nb = EPAD // 128 // ER
    o = pl.pallas_call(
        _edge_idx_kernel,
        grid=(2, nb),
        in_specs=[
            pl.BlockSpec((ER, 128), lambda c, i: (i, 0)),
            pl.BlockSpec((ER, 128), lambda c, i: (i, 0)),
            pl.BlockSpec((1, 1), lambda c, i: (0, 0)),
        ],
        out_specs=[
            pl.BlockSpec((1, ER, 128), lambda c, i: (c, i, 0)),
            pl.BlockSpec((1, ER, 128), lambda c, i: (c, i, 0)),
            pl.BlockSpec((1, ER, 128), lambda c, i: (0, i, 0)),
            pl.BlockSpec((1, ER, 128), lambda c, i: (0, i, 0)),
            pl.BlockSpec((1, ER, 128), lambda c, i: (0, i, 0)),
        ],
        out_shape=[
            jax.ShapeDtypeStruct((2, EPAD // 128, 128), jnp.int32),
            jax.ShapeDtypeStruct((2, EPAD // 128, 128), jnp.int32),
            jax.ShapeDtypeStruct((1, EPAD // 128, 128), jnp.int32),
            jax.ShapeDtypeStruct((1, EPAD // 128, 128), jnp.int32),
            jax.ShapeDtypeStruct((1, EPAD // 128, 128), jnp.int32),
        ],
    )(s2, d2, curr_node_id.reshape(1, 1))
    gid, loc, idx1, sp, dp = o
    return (gid.reshape(2, EPAD // BK, BK), loc.reshape(2, EPAD // BK, BK),
            idx1.reshape(EPAD // BK, BK), sp.reshape(EPAD // BK, BK),
            dp.reshape(EPAD // BK, BK))


MR = NPAD // 16
TSTR = MR // 32
DSTR = MR // 16


DROWS = ROWS_PT // 2


def _deg_sc(dst3, iota_rows):
    mesh = plsc.VectorSubcoreMesh(core_axis_name="c", subcore_axis_name="s")

    @functools.partial(
        pl.kernel,
        mesh=mesh,
        compiler_params=pltpu.CompilerParams(use_tc_tiling_on_sc=False,
                                             needs_layout_passes=False),
        out_type=jax.ShapeDtypeStruct((2, MR, 16), jnp.float32),
        scratch_types=[
            pltpu.VMEM((DROWS, BK), jnp.int32),
            pltpu.VMEM((MR, 16), jnp.float32),
            pltpu.VMEM((5, 128), jnp.int32),
            pltpu.VMEM((DSTR, 16), jnp.float32),
            pltpu.VMEM_SHARED((MR, 16), jnp.float32),
        ],
    )
    def k(dst_hbm, iota_hbm, deg_hbm, d_seg, lhist, iota_v, zrow, shist):
        c = lax.axis_index("c")
        t = lax.axis_index("s")
        ones16 = jnp.ones((16,), jnp.float32)
        zeros16 = jnp.zeros((16,), jnp.float32)

        pltpu.sync_copy(iota_hbm, iota_v)
        w = c * 16 + t
        pltpu.sync_copy(dst_hbm.at[pl.ds(w * DROWS, DROWS)], d_seg)

        def z0(i, _):
            zrow[i, :] = zeros16
            return _
        lax.fori_loop(0, DSTR, z0, None)

        def z1(i, _):
            lhist[i, :] = zeros16
            return _
        lax.fori_loop(0, MR, z1, None)
        pltpu.sync_copy(zrow, shist.at[pl.ds(t * DSTR, DSTR)])
        plsc.subcore_barrier()

        def bodyD(j, _):
            for q in range(BK // 16):
                d16 = d_seg[j, pl.ds(q * 16, 16)]
                md = d16 >= 0
                d16c = jnp.maximum(d16, 0)
                plsc.addupdate_scatter(
                    lhist, [d16c >> 4, d16c & 15], ones16, mask=md)
            return _
        lax.fori_loop(0, DROWS, bodyD, None)

        for b in range(5):
            pltpu.sync_copy(lhist.at[pl.ds(b * 128, 128)],
                            shist.at[plsc.Indices(iota_v.at[b])], add=True)
        plsc.subcore_barrier()

        pltpu.sync_copy(shist.at[pl.ds(t * DSTR, DSTR)],
                        deg_hbm.at[c, pl.ds(t * DSTR, DSTR)])

    return k(dst3, iota_rows)


def _bfs_sc(idx1, src3, dst3, iota_rows, curr_node_id):
    mesh = plsc.VectorSubcoreMesh(core_axis_name="c", subcore_axis_name="s")

    @functools.partial(
        pl.kernel,
        mesh=mesh,
        compiler_params=pltpu.CompilerParams(use_tc_tiling_on_sc=False,
                                             needs_layout_passes=False),
        out_type=jax.ShapeDtypeStruct((MR, 16), jnp.float32),
        scratch_types=[
            pltpu.VMEM((SEGB, BK), jnp.int32),
            pltpu.VMEM((SEGB, BK), jnp.int32),
            pltpu.VMEM((MR, 16), jnp.float32),
            pltpu.VMEM((MR, 16), jnp.float32),
            pltpu.VMEM((5, 128), jnp.int32),
            pltpu.VMEM((DSTR, 16), jnp.float32),
            pltpu.VMEM((16,), jnp.int32),
            pltpu.VMEM_SHARED((MR, 16), jnp.float32),
            pltpu.VMEM_SHARED((MR, 16), jnp.float32),
        ],
    )
    def k(idx1_hbm, src_hbm, dst_hbm, iota_hbm, curr_hbm, nb_hbm,
          a_seg, d_seg, lmask, lhist, iota_v, zrow, curr_v,
          smask1, smask2):
        c = lax.axis_index("c")
        t = lax.axis_index("s")
        ones16 = jnp.ones((16,), jnp.float32)
        zeros16 = jnp.zeros((16,), jnp.float32)

        pltpu.sync_copy(curr_hbm, curr_v)
        pltpu.sync_copy(iota_hbm, iota_v)

        def z0(i, _):
            zrow[i, :] = zeros16
            return _
        lax.fori_loop(0, DSTR, z0, None)

        def z1(i, _):
            lmask[i, :] = zeros16
            return _
        lax.fori_loop(0, MR, z1, None)
        zb = t * DSTR
        pltpu.sync_copy(zrow, smask1.at[pl.ds(zb, DSTR)])
        pltpu.sync_copy(zrow, smask2.at[pl.ds(zb, DSTR)])
        plsc.subcore_barrier()

        def segA(g, _):
            row0 = t * ROWS_PT + g * SEGB
            pltpu.sync_copy(idx1_hbm.at[pl.ds(row0, SEGB)], a_seg)

            def bodyA(j, _):
                for q in range(BK // 16):
                    i16 = a_seg[j, pl.ds(q * 16, 16)]
                    m = i16 >= 0
                    i16c = jnp.maximum(i16, 0)
                    plsc.store_scatter(
                        lmask, [i16c >> 4, i16c & 15], ones16, mask=m)
                return _
            lax.fori_loop(0, SEGB, bodyA, None)
            return _
        lax.fori_loop(0, NSEG, segA, None)

        for b in range(5):
            pltpu.sync_copy(lmask.at[pl.ds(b * 128, 128)],
                            smask1.at[plsc.Indices(iota_v.at[b])], add=True)
        plsc.subcore_barrier()

        cv = curr_v[...]
        pltpu.sync_copy(smask1, lhist)

        def z2(i, _):
            lmask[i, :] = zeros16
            return _
        lax.fori_loop(0, MR, z2, None)

        def segB(g, _):
            row0 = t * ROWS_PT + g * SEGB
            pltpu.sync_copy(src_hbm.at[pl.ds(row0, SEGB)], a_seg)
            pltpu.sync_copy(dst_hbm.at[pl.ds(row0, SEGB)], d_seg)

            def bodyB(j, _):
                for q in range(BK // 16):
                    s16 = a_seg[j, pl.ds(q * 16, 16)]
                    d16 = d_seg[j, pl.ds(q * 16, 16)]
                    d16c = jnp.maximum(d16, 0)
                    mv = plsc.load_gather(lhist, [d16c >> 4, d16c & 15])
                    hit = ((mv > 0.0) | (d16 == cv)) & (d16 >= 0)
                    plsc.store_scatter(
                        lmask, [s16 >> 4, s16 & 15], ones16, mask=hit)
                return _
            lax.fori_loop(0, SEGB, bodyB, None)
            return _
        lax.fori_loop(0, NSEG, segB, None)

        for b in range(5):
            pltpu.sync_copy(lmask.at[pl.ds(b * 128, 128)],
                            smask2.at[plsc.Indices(iota_v.at[b])], add=True)
        plsc.subcore_barrier()

        nbase = (c * 16 + t) * TSTR
        pltpu.sync_copy(smask1.at[pl.ds(nbase, TSTR)], lmask.at[pl.ds(0, TSTR)])
        pltpu.sync_copy(smask2.at[pl.ds(nbase, TSTR)],
                        lmask.at[pl.ds(TSTR, TSTR)])

        def cbody(r, _):
            m1 = lmask[r, :]
            m2 = lmask[TSTR + r, :]
            ids = (nbase + r) * 16 + lax.iota(jnp.int32, 16)
            nb = jnp.where((m1 + m2) > 0.0, 1.0, 0.0)
            nb = jnp.where(ids == cv, 0.0, nb)
            lmask[2 * TSTR + r, :] = nb
            return _
        lax.fori_loop(0, TSTR, cbody, None)
        pltpu.sync_copy(lmask.at[pl.ds(2 * TSTR, TSTR)],
                        nb_hbm.at[pl.ds(nbase, TSTR)])

    return k(idx1, src3, dst3, iota_rows, curr_node_id)


def _mm_scale_kernel(x_ref, w_ref, s_ref, o_ref):
    i = pl.program_id(0)
    rows = i * BM + lax.broadcasted_iota(jnp.int32, (BM, 1), 0)
    xv = jnp.where(rows < N, x_ref[...], 0.0)
    o_ref[...] = jnp.dot(xv, w_ref[...],
                         preferred_element_type=jnp.float32) \
        * lax.rsqrt(s_ref[...] + 1.0)


def _mm_scale(x, w, s):
    return pl.pallas_call(
        _mm_scale_kernel,
        grid=(GRID,),
        in_specs=[
            pl.BlockSpec((BM, D), lambda i: (i, 0)),
            pl.BlockSpec((D, H), lambda i: (0, 0)),
            pl.BlockSpec((BM, 1), lambda i: (i, 0)),
        ],
        out_specs=pl.BlockSpec((BM, H), lambda i: (i, 0)),
        out_shape=jax.ShapeDtypeStruct((NPAD, H), jnp.float32),
    )(x, w, s)


def _agg_stats_kernel(a_ref, y_ref, s_ref, b_ref, z_ref, ps_ref, pq_ref):
    i = pl.program_id(0)
    z = lax.rsqrt(s_ref[...] + 1.0) * (a_ref[...] + y_ref[...]) + b_ref[...]
    rows = i * BM + lax.broadcasted_iota(jnp.int32, (BM, 1), 0)
    z = jnp.where(rows < N, z, 0.0)
    z_ref[...] = z
    ps_ref[...] = jnp.sum(z, axis=0, keepdims=True)[None]
    pq_ref[...] = jnp.sum(z * z, axis=0, keepdims=True)[None]


def _agg_stats(acc, y, dinv, b):
    return pl.pallas_call(
        _agg_stats_kernel,
        grid=(GRID,),
        in_specs=[
            pl.BlockSpec((BM, H), lambda i: (i, 0)),
            pl.BlockSpec((BM, H), lambda i: (i, 0)),
            pl.BlockSpec((BM, 1), lambda i: (i, 0)),
            pl.BlockSpec((1, H), lambda i: (0, 0)),
        ],
        out_specs=[
            pl.BlockSpec((BM, H), lambda i: (i, 0)),
            pl.BlockSpec((1, 1, H), lambda i: (i, 0, 0)),
            pl.BlockSpec((1, 1, H), lambda i: (i, 0, 0)),
        ],
        out_shape=[
            jax.ShapeDtypeStruct((NPAD, H), jnp.float32),
            jax.ShapeDtypeStruct((GRID, 1, H), jnp.float32),
            jax.ShapeDtypeStruct((GRID, 1, H), jnp.float32),
        ],
    )(acc, y, dinv, b)


def _norm_mm_kernel(z_ref, ps_ref, pq_ref, w_ref, s_ref, h_ref, y_ref):
    m = jnp.sum(ps_ref[...], axis=0) / N
    vv = jnp.sum(pq_ref[...], axis=0) / N - m * m
    r = lax.rsqrt(vv + 1e-5)
    hn = jnp.maximum((z_ref[...] - m) * r, 0.0)
    i = pl.program_id(0)
    rows = i * BM + lax.broadcasted_iota(jnp.int32, (BM, 1), 0)
    hn = jnp.where(rows < N, hn, 0.0)
    h_ref[...] = hn
    y_ref[...] = jnp.dot(hn, w_ref[...],
                         preferred_element_type=jnp.float32) \
        * lax.rsqrt(s_ref[...] + 1.0)


def _norm_mm(z, ps, pq, w, s):
    return pl.pallas_call(
        _norm_mm_kernel,
        grid=(GRID,),
        in_specs=[
            pl.BlockSpec((BM, H), lambda i: (i, 0)),
            pl.BlockSpec((GRID, 1, H), lambda i: (0, 0, 0)),
            pl.BlockSpec((GRID, 1, H), lambda i: (0, 0, 0)),
            pl.BlockSpec((H, H), lambda i: (0, 0)),
            pl.BlockSpec((BM, 1), lambda i: (i, 0)),
        ],
        out_specs=[
            pl.BlockSpec((BM, H), lambda i: (i, 0)),
            pl.BlockSpec((BM, H), lambda i: (i, 0)),
        ],
        out_shape=[
            jax.ShapeDtypeStruct((NPAD, H), jnp.float32),
            jax.ShapeDtypeStruct((NPAD, H), jnp.float32),
        ],
    )(z, ps, pq, w, s)


def _mlp_pool_kernel(ha_ref, nw_ref, xc_ref, bmat_ref, b1_ref, v_ref,
                     w2_ref, b2_ref, msk_ref, part_ref, o_ref):
    i = pl.program_id(0)
    c = jnp.dot(xc_ref[...], bmat_ref[...],
                preferred_element_type=jnp.float32) + b1_ref[...]
    s = jnp.maximum(ha_ref[...] + nw_ref[...] * v_ref[...] + c, 0.0)
    sc = (jnp.dot(s, w2_ref[...], preferred_element_type=jnp.float32)
          + b2_ref[...]) * msk_ref[...]
    rows = i * BM + lax.broadcasted_iota(jnp.int32, (BM, 1), 0)
    part = jnp.where(rows < N, part_ref[...], 0.0)
    contrib = jnp.dot(sc.T, part, preferred_element_type=jnp.float32)

    @pl.when(i == 0)
    def _():
        o_ref[...] = jnp.zeros_like(o_ref)
    o_ref[...] += contrib


def _mlp_pool(ha, nw, xc, bmat, b1, v, w2, b2, msk, part):
    return pl.pallas_call(
        _mlp_pool_kernel,
        grid=(GRID,),
        in_specs=[
            pl.BlockSpec((BM, H), lambda i: (i, 0)),
            pl.BlockSpec((BM, 1), lambda i: (i, 0)),
            pl.BlockSpec((1, H), lambda i: (0, 0)),
            pl.BlockSpec((H, H), lambda i: (0, 0)),
            pl.BlockSpec((1, H), lambda i: (0, 0)),
            pl.BlockSpec((1, H), lambda i: (0, 0)),
            pl.BlockSpec((H, 1), lambda i: (0, 0)),
            pl.BlockSpec((1, 1), lambda i: (0, 0)),
            pl.BlockSpec((BM, 1), lambda i: (i, 0)),
            pl.BlockSpec((BM, P), lambda i: (i, 0)),
        ],
        out_specs=pl.BlockSpec((1, P), lambda i: (0, 0)),
        out_shape=jax.ShapeDtypeStruct((1, P), jnp.float32),
    )(ha, nw, xc, bmat, b1, v, w2, b2, msk, part)


def kernel(x, edge_index, curr_node_id, partitions, node_weights,
           W1, b1, W2, b2, lin1_W, lin1_b, lin2_W, lin2_b):
    src = edge_index[0]
    dst = edge_index[1]
    gid, loc, idx1, sp3, dp3 = _edge_idx(src, dst, curr_node_id)

    iota_rows = jnp.arange(MR, dtype=jnp.int32).reshape(5, 128)
    degh = _deg_sc(dp3, iota_rows)
    deg2 = (degh[0] + degh[1]).reshape(NPAD, 1)
    nb2 = _bfs_sc(idx1, sp3, dp3, iota_rows,
                  jnp.broadcast_to(curr_node_id, (16,)))

    def gcn_bn(y_pad, b):
        o = _spmm_sc(y_pad, gid, loc)
        acc = o.reshape(NPAD, H)
        return _agg_stats(acc, y_pad, deg2, b.reshape(1, H))

    zero_s = jnp.zeros((NPAD, 1), jnp.float32)

    y1 = _mm_scale(x, W1, deg2)
    z1, ps1, pq1 = gcn_bn(y1, b1)
    _, y2 = _norm_mm(z1, ps1, pq1, W2, deg2)
    z2, ps2, pq2 = gcn_bn(y2, b2)
    A = lin1_W[:H]
    h_pad, hA = _norm_mm(z2, ps2, pq2, A, zero_s)
    h = h_pad[:N]

    curr = curr_node_id[0]
    x_curr = h[curr_node_id]

    Bmat = lin1_W[H:2 * H]
    v = lin1_W[2 * H].reshape(1, H)
    nw = jnp.pad(node_weights * node_weights[curr], (0, NPAD - N)
                 ).reshape(NPAD, 1)
    msk = nb2.reshape(NPAD, 1)

    partition_scores = _mlp_pool(hA, nw, x_curr, Bmat, lin1_b.reshape(1, H),
                                 v, lin2_W, lin2_b.reshape(1, 1), msk,
                                 partitions)
    return (partition_scores, h)
